# baseline (speedup 1.0000x reference)
import jax
import jax.numpy as jnp
from jax import lax
from jax.experimental import pallas as pl
from jax.experimental.pallas import tpu as pltpu
from jax.experimental.pallas import tpu_sc as plsc

VOCAB = 10001
EMBED = 16
SEQ = 200
BATCH = 16384
VP = 10112
NC, NS, L = 2, 16, 16
NW = NC * NS
RPW = BATCH // NW
RB = 512
GRID = BATCH // RB
CH = 128
NCH = RPW // CH


def _prep_body(idx_ref, table_ref, w_ref, b_ref, re_ref, tw_ref):
    @pl.when(pl.program_id(0) == 0)
    def _():
        s = (jnp.sum(table_ref[...] * w_ref[...], axis=1) + b_ref[0]) * (1.0 / SEQ)
        tw_ref[pl.ds(0, VOCAB)] = s

    blk = idx_ref[...]
    re_ref[0:RB, :] = blk[:, 0:128]
    re_ref[RB:2 * RB, :] = blk[:, SEQ - 128:SEQ]


def _pool_body(tw_hbm, idx_hbm, out_hbm,
               tw_v, l0, l1, r0, r1, out_v, s0, s1, s2, s3):
    wid = lax.axis_index("s") * NC + lax.axis_index("c")
    base = wid * (2 * RPW)
    lbuf, rbuf = (l0, l1), (r0, r1)
    lsem, rsem = (s0, s1), (s2, s3)
    lane = lax.broadcasted_iota(jnp.int32, (L,), 0)
    zero = jnp.zeros((L,), jnp.float32)

    def start(c):
        lcp = pltpu.async_copy(
            idx_hbm.at[pl.ds(base + c * CH, CH)], lbuf[c % 2], lsem[c % 2])
        rcp = pltpu.async_copy(
            idx_hbm.at[pl.ds(base + RPW + c * CH, CH)], rbuf[c % 2], rsem[c % 2])
        return lcp, rcp

    cps = [None, None]
    cps[0] = start(0)
    pltpu.sync_copy(tw_hbm, tw_v)
    for c in range(NCH):
        if c + 1 < NCH:
            cps[(c + 1) % 2] = start(c + 1)
        for cp in cps[c % 2]:
            cp.wait()
        lb, rb = lbuf[c % 2], rbuf[c % 2]

        def row(rr, carry):
            acc = zero
            for j in range(8):
                acc = acc + plsc.load_gather(tw_v, [lb[rr, pl.ds(j * L, L)]])
            for j in range(4, 8):
                acc = acc + plsc.load_gather(tw_v, [rb[rr, pl.ds(j * L, L)]])
            v = plsc.load_gather(tw_v, [rb[rr, pl.ds(3 * L, L)]])
            acc = acc + jnp.where(lane >= 8, v, zero)
            s = jnp.sum(acc)
            plsc.store_scatter(out_v, [lane * 0 + (c * CH + rr)],
                               jnp.where(lane < 1, s, 0.0), mask=lane < 1)
            return carry

        lax.fori_loop(0, CH, row, 0)
    pltpu.sync_copy(out_v, out_hbm.at[pl.ds(wid * RPW, RPW)])


def kernel(inputs, table, dense_w, dense_b):
    idx = inputs.astype(jnp.int32)
    w_row = dense_w.reshape(1, EMBED)
    idx_re, tw = pl.pallas_call(
        _prep_body,
        grid=(GRID,),
        in_specs=[
            pl.BlockSpec((RB, SEQ), lambda i: (i, 0)),
            pl.BlockSpec((VOCAB, EMBED), lambda i: (0, 0)),
            pl.BlockSpec((1, EMBED), lambda i: (0, 0)),
            pl.BlockSpec((1,), lambda i: (0,)),
        ],
        out_specs=[
            pl.BlockSpec((2 * RB, 128), lambda i: (i, 0)),
            pl.BlockSpec((VP,), lambda i: (0,)),
        ],
        out_shape=[
            jax.ShapeDtypeStruct((2 * BATCH, 128), jnp.int32),
            jax.ShapeDtypeStruct((VP,), jnp.float32),
        ],
    )(idx, table, w_row, dense_b.astype(jnp.float32))

    pool = pl.kernel(
        _pool_body,
        out_type=jax.ShapeDtypeStruct((BATCH,), jnp.float32),
        mesh=plsc.VectorSubcoreMesh(core_axis_name="c", subcore_axis_name="s"),
        scratch_types=[
            pltpu.VMEM((VP,), jnp.float32),
            pltpu.VMEM((CH, 128), jnp.int32),
            pltpu.VMEM((CH, 128), jnp.int32),
            pltpu.VMEM((CH, 128), jnp.int32),
            pltpu.VMEM((CH, 128), jnp.int32),
            pltpu.VMEM((RPW,), jnp.float32),
            pltpu.SemaphoreType.DMA,
            pltpu.SemaphoreType.DMA,
            pltpu.SemaphoreType.DMA,
            pltpu.SemaphoreType.DMA,
        ],
        compiler_params=pltpu.CompilerParams(needs_layout_passes=False),
    )
    out = pool(tw, idx_re)
    return out.reshape(BATCH, 1)


# speedup vs baseline: 1.6565x; 1.6565x over previous
import jax
import jax.numpy as jnp
from jax import lax
from jax.experimental import pallas as pl
from jax.experimental.pallas import tpu as pltpu
from jax.experimental.pallas import tpu_sc as plsc

VOCAB = 10001
EMBED = 16
SEQ = 200
BATCH = 16384
VP = 10112
NC, NS, L = 2, 16, 16
NW = NC * NS
RPW = BATCH // NW
KB = RPW // 128
GRID = NW


def _prep_body(idx_ref, table_ref, w_ref, b_ref, re_ref, tw_ref):
    @pl.when(pl.program_id(0) == 0)
    def _():
        s = (jnp.sum(table_ref[...] * w_ref[...], axis=0) + b_ref[0]) * (1.0 / SEQ)
        tw_ref[pl.ds(0, VOCAB)] = s

    for k in range(KB):
        re_ref[k * SEQ:(k + 1) * SEQ, :] = idx_ref[:, k * 128:(k + 1) * 128]


def _pool_body(tw_hbm, idx_hbm, out_hbm, tw_v, b0, b1, out_v, s0, s1):
    wid = lax.axis_index("s") * NC + lax.axis_index("c")
    base = wid * (KB * SEQ)
    buf, sem = (b0, b1), (s0, s1)

    def start(k):
        return pltpu.async_copy(
            idx_hbm.at[pl.ds(base + k * SEQ, SEQ)], buf[k % 2], sem[k % 2])

    cps = [None, None]
    cps[0] = start(0)
    pltpu.sync_copy(tw_hbm, tw_v)
    zero = jnp.zeros((L,), jnp.float32)
    for k in range(KB):
        if k + 1 < KB:
            cps[(k + 1) % 2] = start(k + 1)
        cps[k % 2].wait()
        bk = buf[k % 2]

        def step(l, accs):
            return tuple(
                a + plsc.load_gather(tw_v, [bk[l, pl.ds(g * L, L)]])
                for g, a in enumerate(accs)
            )

        accs = lax.fori_loop(0, SEQ, step, (zero,) * 8)
        for g in range(8):
            out_v[pl.ds(k * 128 + g * L, L)] = accs[g]
    pltpu.sync_copy(out_v, out_hbm.at[pl.ds(wid * RPW, RPW)])


def kernel(inputs, table, dense_w, dense_b):
    idx_t = inputs.astype(jnp.int32).T
    table_t = table.T
    idx_re, tw = pl.pallas_call(
        _prep_body,
        grid=(GRID,),
        in_specs=[
            pl.BlockSpec((SEQ, RPW), lambda i: (0, i)),
            pl.BlockSpec((EMBED, VOCAB), lambda i: (0, 0)),
            pl.BlockSpec((EMBED, 1), lambda i: (0, 0)),
            pl.BlockSpec((1,), lambda i: (0,)),
        ],
        out_specs=[
            pl.BlockSpec((KB * SEQ, 128), lambda i: (i, 0)),
            pl.BlockSpec((VP,), lambda i: (0,)),
        ],
        out_shape=[
            jax.ShapeDtypeStruct((NW * KB * SEQ, 128), jnp.int32),
            jax.ShapeDtypeStruct((VP,), jnp.float32),
        ],
    )(idx_t, table_t, dense_w, dense_b.astype(jnp.float32))

    pool = pl.kernel(
        _pool_body,
        out_type=jax.ShapeDtypeStruct((BATCH,), jnp.float32),
        mesh=plsc.VectorSubcoreMesh(core_axis_name="c", subcore_axis_name="s"),
        scratch_types=[
            pltpu.VMEM((VP,), jnp.float32),
            pltpu.VMEM((SEQ, 128), jnp.int32),
            pltpu.VMEM((SEQ, 128), jnp.int32),
            pltpu.VMEM((RPW,), jnp.float32),
            pltpu.SemaphoreType.DMA,
            pltpu.SemaphoreType.DMA,
        ],
        compiler_params=pltpu.CompilerParams(needs_layout_passes=False),
    )
    out = pool(tw, idx_re)
    return out.reshape(BATCH, 1)
